# Initial kernel scaffold; baseline (speedup 1.0000x reference)
import jax
import jax.numpy as jnp
from jax.experimental import pallas as pl


def kernel(z, codebook):
    raise NotImplementedError("write your pallas kernel here")


# speedup vs baseline: 1.4370x; 1.4370x over previous
import functools

import jax
import jax.numpy as jnp
from jax import lax
from jax.experimental import pallas as pl
from jax.experimental.pallas import tpu as pltpu
from jax.experimental.pallas import tpu_sc as plsc

_NUM_CODES = 512
_DIM = 32


_TOK_BLOCK = 2048


def _argmin_body(z_ref, cb_ref, idx_ref):
    z = z_ref[...]
    cb = cb_ref[...]
    mm = lax.dot_general(z, cb, (((1,), (1,)), ((), ())))
    zsq = jnp.sum(z * z, axis=1, keepdims=True)
    csq = jnp.sum(cb * cb, axis=1)
    d = zsq - 2.0 * mm + csq
    dmin = jnp.min(d, axis=1, keepdims=True)
    ids = lax.broadcasted_iota(jnp.int32, d.shape, 1)
    idx = jnp.min(jnp.where(d == dmin, ids, _NUM_CODES), axis=1)
    idx_ref[...] = idx.reshape(1, 1, -1)


def _argmin_indices(flat_z, codebook):
    n = flat_z.shape[0]
    grid = n // _TOK_BLOCK
    out = pl.pallas_call(
        _argmin_body,
        grid=(grid,),
        in_specs=[
            pl.BlockSpec((_TOK_BLOCK, _DIM), lambda i: (i, 0)),
            pl.BlockSpec((_NUM_CODES, _DIM), lambda i: (0, 0)),
        ],
        out_specs=pl.BlockSpec((1, 1, _TOK_BLOCK), lambda i: (i, 0, 0)),
        out_shape=jax.ShapeDtypeStruct((grid, 1, _TOK_BLOCK), jnp.int32),
    )(flat_z, codebook)
    return out.reshape(-1)


_NC, _NS = 2, 16
_NW = _NC * _NS
_CHUNK = 2048
_LANES = 16


def _make_sc_gather(batch):
    bpw = batch // _NW
    mesh = plsc.VectorSubcoreMesh(core_axis_name="c", subcore_axis_name="s")

    @functools.partial(
        pl.kernel,
        mesh=mesh,
        out_type=jax.ShapeDtypeStruct((batch * _DIM,), jnp.float32),
        scratch_types=[
            pltpu.VMEM((_NUM_CODES * _DIM,), jnp.float32),
            pltpu.VMEM((bpw,), jnp.int32),
            pltpu.VMEM((_CHUNK * _DIM,), jnp.float32),
        ],
        compiler_params=pltpu.CompilerParams(needs_layout_passes=False),
    )
    def sc_gather(table_hbm, idx_hbm, out_hbm, cb_v, idx_v, packed_v):
        wid = lax.axis_index("s") * _NC + lax.axis_index("c")
        base = wid * bpw
        pltpu.sync_copy(table_hbm, cb_v)
        pltpu.sync_copy(idx_hbm.at[pl.ds(base, bpw)], idx_v)
        lane_off = lax.iota(jnp.int32, _LANES) * _DIM

        for c in range(bpw // _CHUNK):
            def body(g, _):
                idx16 = idx_v[pl.ds(c * _CHUNK + g * _LANES, _LANES)]
                src = idx16 * _DIM
                dst = lane_off + g * (_LANES * _DIM)
                for j in range(_DIM):
                    v = plsc.load_gather(cb_v, [src + j])
                    plsc.store_scatter(packed_v, [dst + j], v)
                return 0

            lax.fori_loop(0, _CHUNK // _LANES, body, 0)
            pltpu.sync_copy(
                packed_v,
                out_hbm.at[pl.ds((base + c * _CHUNK) * _DIM, _CHUNK * _DIM)],
            )

    return sc_gather


def kernel(z, codebook):
    flat_z = z.reshape(-1, _DIM)
    indices = _argmin_indices(flat_z, codebook)
    quantized = _make_sc_gather(flat_z.shape[0])(codebook.reshape(-1), indices)
    return quantized.reshape(z.shape)


# speedup vs baseline: 1.9538x; 1.3597x over previous
import functools

import jax
import jax.numpy as jnp
from jax import lax
from jax.experimental import pallas as pl
from jax.experimental.pallas import tpu as pltpu
from jax.experimental.pallas import tpu_sc as plsc

_NUM_CODES = 512
_DIM = 32


_TOK_BLOCK = 2048


def _argmin_body(z_ref, cb_ref, idx_ref):
    z = z_ref[...]
    cb = cb_ref[...]
    mm = lax.dot_general(z, cb, (((1,), (1,)), ((), ())))
    zsq = jnp.sum(z * z, axis=1, keepdims=True)
    csq = jnp.sum(cb * cb, axis=1)
    d = zsq - 2.0 * mm + csq
    dmin = jnp.min(d, axis=1, keepdims=True)
    ids = lax.broadcasted_iota(jnp.int32, d.shape, 1)
    idx = jnp.min(jnp.where(d == dmin, ids, _NUM_CODES), axis=1)
    idx_ref[...] = idx.reshape(1, 1, -1)


def _argmin_indices(flat_z, codebook):
    n = flat_z.shape[0]
    grid = n // _TOK_BLOCK
    out = pl.pallas_call(
        _argmin_body,
        grid=(grid,),
        in_specs=[
            pl.BlockSpec((_TOK_BLOCK, _DIM), lambda i: (i, 0)),
            pl.BlockSpec((_NUM_CODES, _DIM), lambda i: (0, 0)),
        ],
        out_specs=pl.BlockSpec((1, 1, _TOK_BLOCK), lambda i: (i, 0, 0)),
        out_shape=jax.ShapeDtypeStruct((grid, 1, _TOK_BLOCK), jnp.int32),
    )(flat_z, codebook)
    return out.reshape(-1)


_NC, _NS = 2, 16
_NW = _NC * _NS
_CHUNK = 2048
_LANES = 16


def _make_sc_gather(batch):
    bpw = batch // _NW
    mesh = plsc.VectorSubcoreMesh(core_axis_name="c", subcore_axis_name="s")

    @functools.partial(
        pl.kernel,
        mesh=mesh,
        out_type=jax.ShapeDtypeStruct((batch * _DIM,), jnp.float32),
        scratch_types=[
            pltpu.VMEM((_NUM_CODES * _DIM,), jnp.float32),
            pltpu.VMEM((bpw,), jnp.int32),
            pltpu.VMEM((_CHUNK * _DIM,), jnp.float32),
        ],
        compiler_params=pltpu.CompilerParams(needs_layout_passes=False),
    )
    def sc_gather(table_hbm, idx_hbm, out_hbm, cb_v, idx_v, packed_v):
        wid = lax.axis_index("s") * _NC + lax.axis_index("c")
        base = wid * bpw
        pltpu.sync_copy(table_hbm, cb_v)
        pltpu.sync_copy(idx_hbm.at[pl.ds(base, bpw)], idx_v)
        lane = lax.iota(jnp.int32, _LANES)
        lane_off = lane * _DIM

        for c in range(bpw // _CHUNK):
            def body(g, _):
                idx16 = idx_v[pl.ds(c * _CHUNK + g * _LANES, _LANES)]
                dst0 = lane_off + g * (_LANES * _DIM)
                for j in range(_DIM):
                    col = (lane + j) & (_DIM - 1)
                    v = plsc.load_gather(cb_v, [col * _NUM_CODES + idx16])
                    plsc.store_scatter(packed_v, [dst0 + col], v)
                return 0

            lax.fori_loop(0, _CHUNK // _LANES, body, 0)
            pltpu.sync_copy(
                packed_v,
                out_hbm.at[pl.ds((base + c * _CHUNK) * _DIM, _CHUNK * _DIM)],
            )

    return sc_gather


def kernel(z, codebook):
    flat_z = z.reshape(-1, _DIM)
    indices = _argmin_indices(flat_z, codebook)
    quantized = _make_sc_gather(flat_z.shape[0])(
        codebook.T.reshape(-1), indices)
    return quantized.reshape(z.shape)


# speedup vs baseline: 2.0201x; 1.0339x over previous
import functools

import jax
import jax.numpy as jnp
from jax import lax
from jax.experimental import pallas as pl
from jax.experimental.pallas import tpu as pltpu
from jax.experimental.pallas import tpu_sc as plsc

_NUM_CODES = 512
_DIM = 32


_TOK_BLOCK = 2048


def _argmin_body(z_ref, cb_ref, idx_ref):
    z = z_ref[...]
    cb = cb_ref[...]
    mm = lax.dot_general(z, cb, (((1,), (1,)), ((), ())))
    zsq = jnp.sum(z * z, axis=1, keepdims=True)
    csq = jnp.sum(cb * cb, axis=1)
    d = zsq - 2.0 * mm + csq
    dmin = jnp.min(d, axis=1, keepdims=True)
    ids = lax.broadcasted_iota(jnp.int32, d.shape, 1)
    idx = jnp.min(jnp.where(d == dmin, ids, _NUM_CODES), axis=1)
    idx_ref[...] = idx.reshape(1, 1, -1)


def _argmin_indices(flat_z, codebook):
    n = flat_z.shape[0]
    grid = n // _TOK_BLOCK
    out = pl.pallas_call(
        _argmin_body,
        grid=(grid,),
        in_specs=[
            pl.BlockSpec((_TOK_BLOCK, _DIM), lambda i: (i, 0)),
            pl.BlockSpec((_NUM_CODES, _DIM), lambda i: (0, 0)),
        ],
        out_specs=pl.BlockSpec((1, 1, _TOK_BLOCK), lambda i: (i, 0, 0)),
        out_shape=jax.ShapeDtypeStruct((grid, 1, _TOK_BLOCK), jnp.int32),
    )(flat_z, codebook)
    return out.reshape(-1)


_NC, _NS = 2, 16
_NW = _NC * _NS
_CHUNK = 2048
_LANES = 16


def _make_sc_gather(batch):
    bpw = batch // _NW
    mesh = plsc.VectorSubcoreMesh(core_axis_name="c", subcore_axis_name="s")

    @functools.partial(
        pl.kernel,
        mesh=mesh,
        out_type=jax.ShapeDtypeStruct((batch * _DIM,), jnp.float32),
        scratch_types=[
            pltpu.VMEM((_NUM_CODES * _DIM,), jnp.float32),
            pltpu.VMEM((bpw,), jnp.int32),
            pltpu.VMEM((_CHUNK * _DIM,), jnp.float32),
        ],
        compiler_params=pltpu.CompilerParams(needs_layout_passes=False),
    )
    def sc_gather(table_hbm, idx_hbm, out_hbm, cb_v, idx_v, packed_v):
        wid = lax.axis_index("s") * _NC + lax.axis_index("c")
        base = wid * bpw
        pltpu.sync_copy(table_hbm, cb_v)
        pltpu.sync_copy(idx_hbm.at[pl.ds(base, bpw)], idx_v)
        lane = lax.iota(jnp.int32, _LANES)
        lane_off = lane * _DIM

        for c in range(bpw // _CHUNK):
            def body(g, _):
                idx16 = idx_v[pl.ds(c * _CHUNK + g * _LANES, _LANES)]
                dst0 = lane_off + g * (_LANES * _DIM)
                for j in range(_DIM):
                    col = (lane + j) & (_DIM - 1)
                    v = plsc.load_gather(cb_v, [col * _NUM_CODES + idx16])
                    plsc.store_scatter(packed_v, [dst0 + col], v)
                return 0

            lax.fori_loop(0, _CHUNK // _LANES, body, 0)
            pltpu.sync_copy(
                packed_v,
                out_hbm.at[pl.ds((base + c * _CHUNK) * _DIM, _CHUNK * _DIM)],
            )

    return sc_gather


def kernel(z, codebook):
    flat_z = z.reshape(-1, _DIM)
    table = codebook.T.reshape(-1)
    half = flat_z.shape[0] // 2
    gather = _make_sc_gather(half)
    idx0 = _argmin_indices(flat_z[:half], codebook)
    q0 = gather(table, idx0)
    idx1 = _argmin_indices(flat_z[half:], codebook)
    q1 = gather(table, idx1)
    return jnp.concatenate([q0, q1]).reshape(z.shape)
